# Initial kernel scaffold; baseline (speedup 1.0000x reference)
import jax
import jax.numpy as jnp
from jax import lax
from jax.experimental import pallas as pl
from jax.experimental.pallas import tpu as pltpu
from jax.experimental.pallas import tpu_sc as plsc

N = 10000
D = 128
E = 320000

NC = 2
NS = 16
NW = NC * NS

K = 128
CHUNKS = -(-E // (NW * K))
EPT = CHUNKS * K
E_PAD = EPT * NW

N_ACC = 10240
DUMMY = N
RPT = N_ACC // NS

B = 400


def _make_seg_kernel(with_counts):
    out_type = [jax.ShapeDtypeStruct((NC, N_ACC, D), jnp.float32)]
    if with_counts:
        out_type.append(jax.ShapeDtypeStruct((NW, N_ACC), jnp.float32))
    scratch = [
        pltpu.VMEM((K,), jnp.int32),
        pltpu.VMEM((K,), jnp.int32),
        pltpu.VMEM((K, D), jnp.float32),
        pltpu.VMEM((16, D), jnp.float32),
        pltpu.VMEM_SHARED((N_ACC, D), jnp.float32),
        pltpu.SemaphoreType.DMA,
    ]
    if with_counts:
        scratch.append(pltpu.VMEM((N_ACC,), jnp.float32))

    def body(src_hbm, dst_hbm, table_hbm, *rest):
        if with_counts:
            parts, cnts, srcv, dstv, rows, zbuf, acc, sem, cntl = rest
        else:
            parts, srcv, dstv, rows, zbuf, acc, sem = rest
        c = lax.axis_index("c")
        s = lax.axis_index("s")
        zero16 = jnp.zeros((16,), jnp.float32)
        ones16 = jnp.ones((16,), jnp.float32)

        for i in range(16):
            for j in range(D // 16):
                zbuf[i, pl.ds(j * 16, 16)] = zero16

        def zacc(i, carry):
            pltpu.sync_copy(zbuf, acc.at[pl.ds(s * RPT + i * 16, 16), :])
            return carry
        lax.fori_loop(0, RPT // 16, zacc, 0)

        if with_counts:
            def zcnt(i, carry):
                cntl[pl.ds(i * 16, 16)] = zero16
                return carry
            lax.fori_loop(0, N_ACC // 16, zcnt, 0)

        plsc.subcore_barrier()

        base0 = (c * NS + s) * EPT

        def step(t, carry):
            base = base0 + t * K
            pltpu.sync_copy(src_hbm.at[pl.ds(base, K)], srcv)
            pltpu.sync_copy(dst_hbm.at[pl.ds(base, K)], dstv)
            pltpu.async_copy(table_hbm.at[srcv], rows, sem).wait()
            pltpu.sync_copy(rows, acc.at[dstv], add=True)
            if with_counts:
                for j in range(K // 16):
                    plsc.addupdate_scatter(
                        cntl, [dstv[pl.ds(j * 16, 16)]], ones16)
            return carry
        lax.fori_loop(0, CHUNKS, step, 0)

        plsc.subcore_barrier()

        pltpu.sync_copy(acc.at[pl.ds(s * RPT, RPT), :],
                        parts.at[c, pl.ds(s * RPT, RPT), :])
        if with_counts:
            pltpu.sync_copy(cntl, cnts.at[c * NS + s])

    return pl.kernel(
        body,
        out_type=tuple(out_type) if with_counts else out_type[0],
        mesh=plsc.VectorSubcoreMesh(core_axis_name="c", subcore_axis_name="s"),
        scratch_types=scratch,
    )


_seg_with_counts = _make_seg_kernel(True)
_seg_no_counts = _make_seg_kernel(False)


def _mm(a, b):
    return lax.dot_general(a, b, (((1,), (0,)), ((), ())),
                           precision=lax.Precision.HIGHEST,
                           preferred_element_type=jnp.float32)


def _inv_counts(cnt_blk):
    ones = jnp.ones((NW, 1), jnp.float32)
    tot = lax.dot_general(cnt_blk, ones, (((0,), (0,)), ((), ())),
                          precision=lax.Precision.HIGHEST,
                          preferred_element_type=jnp.float32)
    return 1.0 / jnp.maximum(tot, 1.0)


def _layer1_body(x_ref, p_ref, cnt_ref, ws1_ref, wn1_ref, ws2_ref, wn2_ref,
                 b1_ref, hs2_ref, hn2_ref):
    inv = _inv_counts(cnt_ref[...])
    agg = (p_ref[0] + p_ref[1]) * inv
    h1 = _mm(x_ref[...], ws1_ref[...]) + _mm(agg, wn1_ref[...]) + b1_ref[...]
    h1 = jnp.maximum(h1, 0.0)
    hs2_ref[...] = _mm(h1, ws2_ref[...])
    hn2_ref[...] = _mm(h1, wn2_ref[...])


def _layer2_body(hs2_ref, q_ref, cnt_ref, b2_ref, out_ref):
    inv = _inv_counts(cnt_ref[...])
    out_ref[...] = hs2_ref[...] + (q_ref[0] + q_ref[1]) * inv + b2_ref[...]


_row_spec = pl.BlockSpec((B, D), lambda i: (i, 0))
_part_spec = pl.BlockSpec((NC, B, D), lambda i: (0, i, 0))
_cnt_spec = pl.BlockSpec((NW, B), lambda i: (0, i))
_w_spec = pl.BlockSpec((D, D), lambda i: (0, 0))
_b_spec = pl.BlockSpec((1, D), lambda i: (0, 0))

_layer1 = pl.pallas_call(
    _layer1_body,
    grid=(N // B,),
    in_specs=[_row_spec, _part_spec, _cnt_spec,
              _w_spec, _w_spec, _w_spec, _w_spec, _b_spec],
    out_specs=[_row_spec, _row_spec],
    out_shape=[jax.ShapeDtypeStruct((N, D), jnp.float32),
               jax.ShapeDtypeStruct((N, D), jnp.float32)],
)

_layer2 = pl.pallas_call(
    _layer2_body,
    grid=(N // B,),
    in_specs=[_row_spec, _part_spec, _cnt_spec, _b_spec],
    out_specs=_row_spec,
    out_shape=jax.ShapeDtypeStruct((N, D), jnp.float32),
)


def kernel(x, edge_index, W_self1, W_neigh1, b1, W_self2, W_neigh2, b2):
    src = edge_index[0]
    dst = edge_index[1]
    pad = E_PAD - E
    src_p = jnp.concatenate([src, jnp.zeros((pad,), jnp.int32)])
    dst_p = jnp.concatenate([dst, jnp.full((pad,), DUMMY, jnp.int32)])

    seg1, cnts = _seg_with_counts(src_p, dst_p, x)
    hs2, hn2 = _layer1(x, seg1, cnts, W_self1, W_neigh1, W_self2, W_neigh2,
                       b1.reshape(1, D))
    seg2 = _seg_no_counts(src_p, dst_p, hn2)
    return _layer2(hs2, seg2, cnts, b2.reshape(1, D))


# speedup vs baseline: 3.8515x; 3.8515x over previous
import jax
import jax.numpy as jnp
from jax import lax
from jax.experimental import pallas as pl
from jax.experimental.pallas import tpu as pltpu
from jax.experimental.pallas import tpu_sc as plsc

N = 10000
D = 128
E = 320000

NC = 2
NS = 16
NW = NC * NS

K = 128
CHUNKS = -(-E // (NW * K))
EPT = CHUNKS * K
E_PAD = EPT * NW

N_ACC = 10240
DUMMY = N
RPT = N_ACC // NS

B = 512


def _make_cnt_kernel():
    scratch = [
        pltpu.VMEM((K,), jnp.int32),
        pltpu.VMEM((K, D), jnp.float32),
        pltpu.VMEM((16, D), jnp.float32),
        pltpu.VMEM_SHARED((N_ACC, D), jnp.float32),
    ]

    def body(dst_hbm, cnts, dstv, obuf, zbuf, cacc):
        c = lax.axis_index("c")
        s = lax.axis_index("s")
        zero16 = jnp.zeros((16,), jnp.float32)
        ones16 = jnp.ones((16,), jnp.float32)
        for i in range(K):
            for j in range(D // 16):
                obuf[i, pl.ds(j * 16, 16)] = ones16
        for i in range(16):
            for j in range(D // 16):
                zbuf[i, pl.ds(j * 16, 16)] = zero16

        for i in range(RPT // 16):
            pltpu.sync_copy(zbuf, cacc.at[pl.ds(s * RPT + i * 16, 16), :])

        plsc.subcore_barrier()

        base0 = (c * NS + s) * EPT
        for t in range(CHUNKS):
            pltpu.sync_copy(dst_hbm.at[pl.ds(base0 + t * K, K)], dstv)
            pltpu.sync_copy(obuf, cacc.at[dstv], add=True)

        plsc.subcore_barrier()

        for i in range(RPT // K):
            r0 = s * RPT + i * K
            pltpu.sync_copy(cacc.at[pl.ds(r0, K), :], obuf)
            pltpu.sync_copy(obuf, cnts.at[c, pl.ds(r0, K), :])

    return pl.kernel(
        body,
        out_type=jax.ShapeDtypeStruct((NC, N_ACC, D), jnp.float32),
        mesh=plsc.VectorSubcoreMesh(core_axis_name="c", subcore_axis_name="s"),
        scratch_types=scratch,
    )


def _make_seg_kernel():
    scratch = [
        pltpu.VMEM((K,), jnp.int32),
        pltpu.VMEM((K,), jnp.int32),
        pltpu.VMEM((K, D), jnp.float32),
        pltpu.VMEM((16, D), jnp.float32),
        pltpu.VMEM_SHARED((N_ACC, D), jnp.float32),
        pltpu.SemaphoreType.DMA,
    ]

    def body(src_hbm, dst_hbm, table_hbm, parts, srcv, dstv, rows, zbuf,
             acc, sem):
        c = lax.axis_index("c")
        s = lax.axis_index("s")
        zero16 = jnp.zeros((16,), jnp.float32)

        for i in range(16):
            for j in range(D // 16):
                zbuf[i, pl.ds(j * 16, 16)] = zero16

        base0 = (c * NS + s) * EPT
        for i in range(RPT // 16):
            pltpu.sync_copy(zbuf, acc.at[pl.ds(s * RPT + i * 16, 16), :])

        plsc.subcore_barrier()

        for t in range(CHUNKS):
            base = base0 + t * K
            pltpu.sync_copy(src_hbm.at[pl.ds(base, K)], srcv)
            pltpu.sync_copy(dst_hbm.at[pl.ds(base, K)], dstv)
            pltpu.async_copy(table_hbm.at[srcv], rows, sem).wait()
            pltpu.sync_copy(rows, acc.at[dstv], add=True)

        plsc.subcore_barrier()

        for i in range(RPT // K):
            r0 = s * RPT + i * K
            pltpu.sync_copy(acc.at[pl.ds(r0, K), :], rows)
            pltpu.sync_copy(rows, parts.at[c, pl.ds(r0, K), :])

    return pl.kernel(
        body,
        out_type=jax.ShapeDtypeStruct((NC, N_ACC, D), jnp.float32),
        mesh=plsc.VectorSubcoreMesh(core_axis_name="c", subcore_axis_name="s"),
        scratch_types=scratch,
    )


_seg_sum = _make_seg_kernel()
_cnt_sum = _make_cnt_kernel()


def _mm(a, b):
    return lax.dot_general(a, b, (((1,), (0,)), ((), ())),
                           precision=lax.Precision.HIGHEST,
                           preferred_element_type=jnp.float32)


def _inv_counts(cnt_blk):
    tot = cnt_blk[0, :, 0:1] + cnt_blk[1, :, 0:1]
    return 1.0 / jnp.maximum(tot, 1.0)


def _layer1_body(x_ref, p_ref, cnt_ref, ws1_ref, wn1_ref, ws2_ref, wn2_ref,
                 b1_ref, hs2_ref, hn2_ref):
    inv = _inv_counts(cnt_ref[...])
    agg = (p_ref[0] + p_ref[1]) * inv
    h1 = _mm(x_ref[...], ws1_ref[...]) + _mm(agg, wn1_ref[...]) + b1_ref[...]
    h1 = jnp.maximum(h1, 0.0)
    hs2_ref[...] = _mm(h1, ws2_ref[...])
    hn2_ref[...] = _mm(h1, wn2_ref[...])


def _layer2_body(hs2_ref, q_ref, cnt_ref, b2_ref, out_ref):
    inv = _inv_counts(cnt_ref[...])
    out_ref[...] = hs2_ref[...] + (q_ref[0] + q_ref[1]) * inv + b2_ref[...]


_row_spec = pl.BlockSpec((B, D), lambda i: (i, 0))
_part_spec = pl.BlockSpec((NC, B, D), lambda i: (0, i, 0))
_cnt_spec = pl.BlockSpec((NC, B, D), lambda i: (0, i, 0))
_w_spec = pl.BlockSpec((D, D), lambda i: (0, 0))
_b_spec = pl.BlockSpec((1, D), lambda i: (0, 0))

_layer1 = pl.pallas_call(
    _layer1_body,
    grid=(pl.cdiv(N, B),),
    in_specs=[_row_spec, _part_spec, _cnt_spec,
              _w_spec, _w_spec, _w_spec, _w_spec, _b_spec],
    out_specs=[_row_spec, _row_spec],
    out_shape=[jax.ShapeDtypeStruct((N, D), jnp.float32),
               jax.ShapeDtypeStruct((N, D), jnp.float32)],
)

_layer2 = pl.pallas_call(
    _layer2_body,
    grid=(pl.cdiv(N, B),),
    in_specs=[_row_spec, _part_spec, _cnt_spec, _b_spec],
    out_specs=_row_spec,
    out_shape=jax.ShapeDtypeStruct((N, D), jnp.float32),
)


def kernel(x, edge_index, W_self1, W_neigh1, b1, W_self2, W_neigh2, b2):
    src = edge_index[0]
    dst = edge_index[1]
    pad = E_PAD - E
    src_p = jnp.concatenate([src, jnp.zeros((pad,), jnp.int32)])
    dst_p = jnp.concatenate([dst, jnp.full((pad,), DUMMY, jnp.int32)])

    cnts = _cnt_sum(dst_p)
    seg1 = _seg_sum(src_p, dst_p, x)
    hs2, hn2 = _layer1(x, seg1, cnts, W_self1, W_neigh1, W_self2, W_neigh2,
                       b1.reshape(1, D))
    seg2 = _seg_sum(src_p, dst_p, hn2)
    return _layer2(hs2, seg2, cnts, b2.reshape(1, D))


# speedup vs baseline: 5.4654x; 1.4190x over previous
import jax
import jax.numpy as jnp
from jax import lax
from jax.experimental import pallas as pl
from jax.experimental.pallas import tpu as pltpu
from jax.experimental.pallas import tpu_sc as plsc

N = 10000
D = 128
E = 320000

NC = 2
NS = 16
NW = NC * NS

K = 128
CHUNKS = -(-E // (NW * K))
EPT = CHUNKS * K
E_PAD = EPT * NW

N_ACC = 10240
DUMMY = N
RPT = N_ACC // NS

B = 512


def _make_cnt_kernel():
    scratch = [
        [pltpu.VMEM((K,), jnp.int32) for _ in range(4)],
        pltpu.VMEM((K, D), jnp.float32),
        pltpu.VMEM((16, D), jnp.float32),
        pltpu.VMEM_SHARED((N_ACC, D), jnp.float32),
        [pltpu.SemaphoreType.DMA for _ in range(4)],
        [pltpu.SemaphoreType.DMA for _ in range(2)],
    ]

    def body(dst_hbm, cnts, dstv, obuf, zbuf, cacc, jsems, ssems):
        c = lax.axis_index("c")
        s = lax.axis_index("s")
        zero16 = jnp.zeros((16,), jnp.float32)
        ones16 = jnp.ones((16,), jnp.float32)
        for i in range(K):
            for j in range(D // 16):
                obuf[i, pl.ds(j * 16, 16)] = ones16
        for i in range(16):
            for j in range(D // 16):
                zbuf[i, pl.ds(j * 16, 16)] = zero16

        base0 = (c * NS + s) * EPT
        jdesc = [None] * CHUNKS
        sdesc = [None] * CHUNKS

        def load_idx(t):
            r = t % 4
            jdesc[t] = pltpu.async_copy(
                dst_hbm.at[pl.ds(base0 + t * K, K)], dstv[r], jsems[r])

        load_idx(0)
        load_idx(1)

        for i in range(RPT // 16):
            pltpu.sync_copy(zbuf, cacc.at[pl.ds(s * RPT + i * 16, 16), :])

        plsc.subcore_barrier()

        for t in range(CHUNKS):
            jdesc[t].wait()
            if t >= 2:
                sdesc[t - 2].wait()
            sdesc[t] = pltpu.async_copy(obuf, cacc.at[dstv[t % 4]],
                                        ssems[t % 2], add=True)
            if t + 2 < CHUNKS:
                load_idx(t + 2)
        sdesc[CHUNKS - 2].wait()
        sdesc[CHUNKS - 1].wait()

        plsc.subcore_barrier()

        for i in range(RPT // K):
            r0 = s * RPT + i * K
            pltpu.sync_copy(cacc.at[pl.ds(r0, K), :], obuf)
            pltpu.sync_copy(obuf, cnts.at[c, pl.ds(r0, K), :])

    return pl.kernel(
        body,
        out_type=jax.ShapeDtypeStruct((NC, N_ACC, D), jnp.float32),
        mesh=plsc.VectorSubcoreMesh(core_axis_name="c", subcore_axis_name="s"),
        scratch_types=scratch,
    )


def _make_seg_kernel():
    scratch = [
        [pltpu.VMEM((K,), jnp.int32) for _ in range(4)],
        [pltpu.VMEM((K,), jnp.int32) for _ in range(4)],
        [pltpu.VMEM((K, D), jnp.float32) for _ in range(2)],
        pltpu.VMEM((16, D), jnp.float32),
        pltpu.VMEM_SHARED((N_ACC, D), jnp.float32),
        [pltpu.SemaphoreType.DMA for _ in range(4)],
        [pltpu.SemaphoreType.DMA for _ in range(4)],
        [pltpu.SemaphoreType.DMA for _ in range(2)],
        [pltpu.SemaphoreType.DMA for _ in range(2)],
    ]

    def body(src_hbm, dst_hbm, table_hbm, parts, srcv, dstv, rowbufs, zbuf,
             acc, isems, jsems, gsems, ssems):
        c = lax.axis_index("c")
        s = lax.axis_index("s")
        zero16 = jnp.zeros((16,), jnp.float32)

        for i in range(16):
            for j in range(D // 16):
                zbuf[i, pl.ds(j * 16, 16)] = zero16

        base0 = (c * NS + s) * EPT

        idesc = [None] * CHUNKS
        jdesc = [None] * CHUNKS
        gdesc = [None] * CHUNKS
        sdesc = [None] * CHUNKS

        def load_idx(t):
            r = t % 4
            idesc[t] = pltpu.async_copy(
                src_hbm.at[pl.ds(base0 + t * K, K)], srcv[r], isems[r])
            jdesc[t] = pltpu.async_copy(
                dst_hbm.at[pl.ds(base0 + t * K, K)], dstv[r], jsems[r])

        load_idx(0)
        load_idx(1)

        for i in range(RPT // 16):
            pltpu.sync_copy(zbuf, acc.at[pl.ds(s * RPT + i * 16, 16), :])

        plsc.subcore_barrier()

        for t in range(CHUNKS):
            b = t % 2
            idesc[t].wait()
            jdesc[t].wait()
            if t >= 2:
                sdesc[t - 2].wait()
            gdesc[t] = pltpu.async_copy(table_hbm.at[srcv[t % 4]],
                                        rowbufs[b], gsems[b])
            if t + 2 < CHUNKS:
                load_idx(t + 2)
            if t >= 1:
                gdesc[t - 1].wait()
                sdesc[t - 1] = pltpu.async_copy(
                    rowbufs[1 - b], acc.at[dstv[(t - 1) % 4]],
                    ssems[1 - b], add=True)
        last = CHUNKS - 1
        gdesc[last].wait()
        sdesc[last] = pltpu.async_copy(rowbufs[last % 2],
                                       acc.at[dstv[last % 4]],
                                       ssems[last % 2], add=True)
        sdesc[last - 1].wait()
        sdesc[last].wait()

        plsc.subcore_barrier()

        nout = RPT // K
        wdesc = [None] * nout
        for i in range(nout):
            b = i % 2
            r0 = s * RPT + i * K
            if i >= 2:
                wdesc[i - 2].wait()
            pltpu.sync_copy(acc.at[pl.ds(r0, K), :], rowbufs[b])
            wdesc[i] = pltpu.async_copy(rowbufs[b],
                                        parts.at[c, pl.ds(r0, K), :],
                                        gsems[b])
        for i in range(max(nout - 2, 0), nout):
            wdesc[i].wait()

    return pl.kernel(
        body,
        out_type=jax.ShapeDtypeStruct((NC, N_ACC, D), jnp.float32),
        mesh=plsc.VectorSubcoreMesh(core_axis_name="c", subcore_axis_name="s"),
        scratch_types=scratch,
    )


_seg_sum = _make_seg_kernel()
_cnt_sum = _make_cnt_kernel()


def _mm(a, b):
    return lax.dot_general(a, b, (((1,), (0,)), ((), ())),
                           precision=lax.Precision.HIGHEST,
                           preferred_element_type=jnp.float32)


def _inv_counts(cnt_blk):
    tot = cnt_blk[0, :, 0:1] + cnt_blk[1, :, 0:1]
    return 1.0 / jnp.maximum(tot, 1.0)


def _layer1_body(x_ref, p_ref, cnt_ref, ws1_ref, wn1_ref, ws2_ref, wn2_ref,
                 b1_ref, hs2_ref, hn2_ref):
    inv = _inv_counts(cnt_ref[...])
    agg = (p_ref[0] + p_ref[1]) * inv
    h1 = _mm(x_ref[...], ws1_ref[...]) + _mm(agg, wn1_ref[...]) + b1_ref[...]
    h1 = jnp.maximum(h1, 0.0)
    hs2_ref[...] = _mm(h1, ws2_ref[...])
    hn2_ref[...] = _mm(h1, wn2_ref[...])


def _layer2_body(hs2_ref, q_ref, cnt_ref, b2_ref, out_ref):
    inv = _inv_counts(cnt_ref[...])
    out_ref[...] = hs2_ref[...] + (q_ref[0] + q_ref[1]) * inv + b2_ref[...]


_row_spec = pl.BlockSpec((B, D), lambda i: (i, 0))
_part_spec = pl.BlockSpec((NC, B, D), lambda i: (0, i, 0))
_cnt_spec = pl.BlockSpec((NC, B, D), lambda i: (0, i, 0))
_w_spec = pl.BlockSpec((D, D), lambda i: (0, 0))
_b_spec = pl.BlockSpec((1, D), lambda i: (0, 0))

_layer1 = pl.pallas_call(
    _layer1_body,
    grid=(pl.cdiv(N, B),),
    in_specs=[_row_spec, _part_spec, _cnt_spec,
              _w_spec, _w_spec, _w_spec, _w_spec, _b_spec],
    out_specs=[_row_spec, _row_spec],
    out_shape=[jax.ShapeDtypeStruct((N, D), jnp.float32),
               jax.ShapeDtypeStruct((N, D), jnp.float32)],
)

_layer2 = pl.pallas_call(
    _layer2_body,
    grid=(pl.cdiv(N, B),),
    in_specs=[_row_spec, _part_spec, _cnt_spec, _b_spec],
    out_specs=_row_spec,
    out_shape=jax.ShapeDtypeStruct((N, D), jnp.float32),
)


def kernel(x, edge_index, W_self1, W_neigh1, b1, W_self2, W_neigh2, b2):
    src = edge_index[0]
    dst = edge_index[1]
    pad = E_PAD - E
    src_p = jnp.concatenate([src, jnp.zeros((pad,), jnp.int32)])
    dst_p = jnp.concatenate([dst, jnp.full((pad,), DUMMY, jnp.int32)])

    cnts = _cnt_sum(dst_p)
    seg1 = _seg_sum(src_p, dst_p, x)
    hs2, hn2 = _layer1(x, seg1, cnts, W_self1, W_neigh1, W_self2, W_neigh2,
                       b1.reshape(1, D))
    seg2 = _seg_sum(src_p, dst_p, hn2)
    return _layer2(hs2, seg2, cnts, b2.reshape(1, D))


# speedup vs baseline: 5.8172x; 1.0644x over previous
import jax
import jax.numpy as jnp
from jax import lax
from jax.experimental import pallas as pl
from jax.experimental.pallas import tpu as pltpu
from jax.experimental.pallas import tpu_sc as plsc

N = 10000
D = 128
E = 320000

NC = 2
NS = 16
NW = NC * NS

K = 128
CHUNKS = -(-E // (NW * K))
EPT = CHUNKS * K
E_PAD = EPT * NW

CH_BIG = 112
CH_SML = 46
BIG_CORE = 0

N_ACC = 10240
DUMMY = N
RPT = N_ACC // NS

B = 512


def _make_cnt_kernel():
    scratch = [
        [pltpu.VMEM((K,), jnp.int32) for _ in range(4)],
        pltpu.VMEM((K, D), jnp.float32),
        pltpu.VMEM((16, D), jnp.float32),
        pltpu.VMEM_SHARED((N_ACC, D), jnp.float32),
        [pltpu.SemaphoreType.DMA for _ in range(4)],
        [pltpu.SemaphoreType.DMA for _ in range(2)],
    ]

    def body(dst_hbm, cnts, dstv, obuf, zbuf, cacc, jsems, ssems):
        c = lax.axis_index("c")
        s = lax.axis_index("s")
        zero16 = jnp.zeros((16,), jnp.float32)
        ones16 = jnp.ones((16,), jnp.float32)
        for i in range(K):
            for j in range(D // 16):
                obuf[i, pl.ds(j * 16, 16)] = ones16
        for i in range(16):
            for j in range(D // 16):
                zbuf[i, pl.ds(j * 16, 16)] = zero16

        base0 = (c * NS + s) * EPT
        jdesc = [None] * CHUNKS
        sdesc = [None] * CHUNKS

        def load_idx(t):
            r = t % 4
            jdesc[t] = pltpu.async_copy(
                dst_hbm.at[pl.ds(base0 + t * K, K)], dstv[r], jsems[r])

        load_idx(0)
        load_idx(1)

        for i in range(RPT // 16):
            pltpu.sync_copy(zbuf, cacc.at[pl.ds(s * RPT + i * 16, 16), :])

        plsc.subcore_barrier()

        for t in range(CHUNKS):
            jdesc[t].wait()
            if t >= 2:
                sdesc[t - 2].wait()
            sdesc[t] = pltpu.async_copy(obuf, cacc.at[dstv[t % 4]],
                                        ssems[t % 2], add=True)
            if t + 2 < CHUNKS:
                load_idx(t + 2)
        sdesc[CHUNKS - 2].wait()
        sdesc[CHUNKS - 1].wait()

        plsc.subcore_barrier()

        for i in range(RPT // K):
            r0 = s * RPT + i * K
            pltpu.sync_copy(cacc.at[pl.ds(r0, K), :], obuf)
            pltpu.sync_copy(obuf, cnts.at[c, pl.ds(r0, K), :])

    return pl.kernel(
        body,
        out_type=jax.ShapeDtypeStruct((NC, N_ACC, D), jnp.float32),
        mesh=plsc.VectorSubcoreMesh(core_axis_name="c", subcore_axis_name="s"),
        scratch_types=scratch,
    )


def _make_seg_kernel():
    scratch = [
        [pltpu.VMEM((K,), jnp.int32) for _ in range(4)],
        [pltpu.VMEM((K,), jnp.int32) for _ in range(4)],
        [pltpu.VMEM((K, D), jnp.float32) for _ in range(2)],
        pltpu.VMEM((16, D), jnp.float32),
        pltpu.VMEM_SHARED((N_ACC, D), jnp.float32),
        [pltpu.SemaphoreType.DMA for _ in range(4)],
        [pltpu.SemaphoreType.DMA for _ in range(4)],
        [pltpu.SemaphoreType.DMA for _ in range(2)],
        [pltpu.SemaphoreType.DMA for _ in range(2)],
    ]

    def body(src_hbm, dst_hbm, table_hbm, parts, srcv, dstv, rowbufs, zbuf,
             acc, isems, jsems, gsems, ssems):
        c = lax.axis_index("c")
        s = lax.axis_index("s")
        zero16 = jnp.zeros((16,), jnp.float32)

        for i in range(16):
            for j in range(D // 16):
                zbuf[i, pl.ds(j * 16, 16)] = zero16

        base0 = jnp.where(c == BIG_CORE,
                          s * (CH_BIG * K),
                          NS * (CH_BIG * K) + s * (CH_SML * K))

        idesc = [None] * CH_BIG
        jdesc = [None] * CH_BIG
        gdesc = [None] * CH_BIG
        sdesc = [None] * CH_BIG

        def load_idx(t):
            r = t % 4
            idesc[t] = pltpu.async_copy(
                src_hbm.at[pl.ds(base0 + t * K, K)], srcv[r], isems[r])
            jdesc[t] = pltpu.async_copy(
                dst_hbm.at[pl.ds(base0 + t * K, K)], dstv[r], jsems[r])

        load_idx(0)
        load_idx(1)

        for i in range(RPT // 16):
            pltpu.sync_copy(zbuf, acc.at[pl.ds(s * RPT + i * 16, 16), :])

        plsc.subcore_barrier()

        def iteration(t, bound):
            b = t % 2
            idesc[t].wait()
            jdesc[t].wait()
            if t >= 2:
                sdesc[t - 2].wait()
            gdesc[t] = pltpu.async_copy(table_hbm.at[srcv[t % 4]],
                                        rowbufs[b], gsems[b])
            if t + 2 < bound:
                load_idx(t + 2)
            if t >= 1:
                gdesc[t - 1].wait()
                sdesc[t - 1] = pltpu.async_copy(
                    rowbufs[1 - b], acc.at[dstv[(t - 1) % 4]],
                    ssems[1 - b], add=True)

        def epilogue(n):
            last = n - 1
            gdesc[last].wait()
            sdesc[last] = pltpu.async_copy(rowbufs[last % 2],
                                           acc.at[dstv[last % 4]],
                                           ssems[last % 2], add=True)
            sdesc[last - 1].wait()
            sdesc[last].wait()

        for t in range(CH_SML):
            iteration(t, CH_SML)

        @pl.when(c == BIG_CORE)
        def _big_core_tail():
            load_idx(CH_SML)
            load_idx(CH_SML + 1)
            for t in range(CH_SML, CH_BIG):
                iteration(t, CH_BIG)
            epilogue(CH_BIG)

        @pl.when(c != BIG_CORE)
        def _small_core_tail():
            epilogue(CH_SML)

        plsc.subcore_barrier()

        nout = RPT // K
        wdesc = [None] * nout
        for i in range(nout):
            b = i % 2
            r0 = s * RPT + i * K
            if i >= 2:
                wdesc[i - 2].wait()
            pltpu.sync_copy(acc.at[pl.ds(r0, K), :], rowbufs[b])
            wdesc[i] = pltpu.async_copy(rowbufs[b],
                                        parts.at[c, pl.ds(r0, K), :],
                                        gsems[b])
        for i in range(max(nout - 2, 0), nout):
            wdesc[i].wait()

    return pl.kernel(
        body,
        out_type=jax.ShapeDtypeStruct((NC, N_ACC, D), jnp.float32),
        mesh=plsc.VectorSubcoreMesh(core_axis_name="c", subcore_axis_name="s"),
        scratch_types=scratch,
    )


_seg_sum = _make_seg_kernel()
_cnt_sum = _make_cnt_kernel()


def _mm(a, b):
    return lax.dot_general(a, b, (((1,), (0,)), ((), ())),
                           precision=lax.Precision.HIGHEST,
                           preferred_element_type=jnp.float32)


def _inv_counts(cnt_blk):
    tot = cnt_blk[0, :, 0:1] + cnt_blk[1, :, 0:1]
    return 1.0 / jnp.maximum(tot, 1.0)


def _layer1_body(x_ref, p_ref, cnt_ref, ws1_ref, wn1_ref, ws2_ref, wn2_ref,
                 b1_ref, hs2_ref, hn2_ref):
    inv = _inv_counts(cnt_ref[...])
    agg = (p_ref[0] + p_ref[1]) * inv
    h1 = _mm(x_ref[...], ws1_ref[...]) + _mm(agg, wn1_ref[...]) + b1_ref[...]
    h1 = jnp.maximum(h1, 0.0)
    hs2_ref[...] = _mm(h1, ws2_ref[...])
    hn2_ref[...] = _mm(h1, wn2_ref[...])


def _layer2_body(hs2_ref, q_ref, cnt_ref, b2_ref, out_ref):
    inv = _inv_counts(cnt_ref[...])
    out_ref[...] = hs2_ref[...] + (q_ref[0] + q_ref[1]) * inv + b2_ref[...]


_row_spec = pl.BlockSpec((B, D), lambda i: (i, 0))
_part_spec = pl.BlockSpec((NC, B, D), lambda i: (0, i, 0))
_cnt_spec = pl.BlockSpec((NC, B, D), lambda i: (0, i, 0))
_w_spec = pl.BlockSpec((D, D), lambda i: (0, 0))
_b_spec = pl.BlockSpec((1, D), lambda i: (0, 0))

_layer1 = pl.pallas_call(
    _layer1_body,
    grid=(pl.cdiv(N, B),),
    in_specs=[_row_spec, _part_spec, _cnt_spec,
              _w_spec, _w_spec, _w_spec, _w_spec, _b_spec],
    out_specs=[_row_spec, _row_spec],
    out_shape=[jax.ShapeDtypeStruct((N, D), jnp.float32),
               jax.ShapeDtypeStruct((N, D), jnp.float32)],
)

_layer2 = pl.pallas_call(
    _layer2_body,
    grid=(pl.cdiv(N, B),),
    in_specs=[_row_spec, _part_spec, _cnt_spec, _b_spec],
    out_specs=_row_spec,
    out_shape=jax.ShapeDtypeStruct((N, D), jnp.float32),
)


def kernel(x, edge_index, W_self1, W_neigh1, b1, W_self2, W_neigh2, b2):
    src = edge_index[0]
    dst = edge_index[1]
    pad = E_PAD - E
    src_p = jnp.concatenate([src, jnp.zeros((pad,), jnp.int32)])
    dst_p = jnp.concatenate([dst, jnp.full((pad,), DUMMY, jnp.int32)])

    cnts = _cnt_sum(dst_p)
    seg1 = _seg_sum(src_p, dst_p, x)
    hs2, hn2 = _layer1(x, seg1, cnts, W_self1, W_neigh1, W_self2, W_neigh2,
                       b1.reshape(1, D))
    seg2 = _seg_sum(src_p, dst_p, hn2)
    return _layer2(hs2, seg2, cnts, b2.reshape(1, D))


# speedup vs baseline: 5.8752x; 1.0100x over previous
import jax
import jax.numpy as jnp
from jax import lax
from jax.experimental import pallas as pl
from jax.experimental.pallas import tpu as pltpu
from jax.experimental.pallas import tpu_sc as plsc

N = 10000
D = 128
E = 320000

NC = 2
NS = 16
NW = NC * NS

K = 128
CHUNKS = -(-E // (NW * K))
EPT = CHUNKS * K
E_PAD = EPT * NW

CH_BIG = 112
CH_SML = 46
BIG_CORE = 1

N_ACC = 10240
DUMMY = N
RPT = N_ACC // NS

B = 512


def _make_cnt_kernel():
    scratch = [
        [pltpu.VMEM((K,), jnp.int32) for _ in range(4)],
        pltpu.VMEM((K, D), jnp.float32),
        pltpu.VMEM((16, D), jnp.float32),
        pltpu.VMEM_SHARED((N_ACC, D), jnp.float32),
        [pltpu.SemaphoreType.DMA for _ in range(4)],
        [pltpu.SemaphoreType.DMA for _ in range(2)],
    ]

    def body(dst_hbm, cnts, dstv, obuf, zbuf, cacc, jsems, ssems):
        c = lax.axis_index("c")
        s = lax.axis_index("s")
        zero16 = jnp.zeros((16,), jnp.float32)
        ones16 = jnp.ones((16,), jnp.float32)
        for i in range(K):
            for j in range(D // 16):
                obuf[i, pl.ds(j * 16, 16)] = ones16
        for i in range(16):
            for j in range(D // 16):
                zbuf[i, pl.ds(j * 16, 16)] = zero16

        base0 = (c * NS + s) * EPT
        jdesc = [None] * CHUNKS
        sdesc = [None] * CHUNKS

        def load_idx(t):
            r = t % 4
            jdesc[t] = pltpu.async_copy(
                dst_hbm.at[pl.ds(base0 + t * K, K)], dstv[r], jsems[r])

        load_idx(0)
        load_idx(1)

        for i in range(RPT // 16):
            pltpu.sync_copy(zbuf, cacc.at[pl.ds(s * RPT + i * 16, 16), :])

        plsc.subcore_barrier()

        for t in range(CHUNKS):
            jdesc[t].wait()
            if t >= 2:
                sdesc[t - 2].wait()
            sdesc[t] = pltpu.async_copy(obuf, cacc.at[dstv[t % 4]],
                                        ssems[t % 2], add=True)
            if t + 2 < CHUNKS:
                load_idx(t + 2)
        sdesc[CHUNKS - 2].wait()
        sdesc[CHUNKS - 1].wait()

        plsc.subcore_barrier()

        for i in range(RPT // K):
            r0 = s * RPT + i * K
            pltpu.sync_copy(cacc.at[pl.ds(r0, K), :], obuf)
            pltpu.sync_copy(obuf, cnts.at[c, pl.ds(r0, K), :])

    return pl.kernel(
        body,
        out_type=jax.ShapeDtypeStruct((NC, N_ACC, D), jnp.float32),
        mesh=plsc.VectorSubcoreMesh(core_axis_name="c", subcore_axis_name="s"),
        scratch_types=scratch,
    )


def _make_seg_kernel():
    scratch = [
        [pltpu.VMEM((K,), jnp.int32) for _ in range(4)],
        [pltpu.VMEM((K,), jnp.int32) for _ in range(4)],
        [pltpu.VMEM((K, D), jnp.float32) for _ in range(2)],
        pltpu.VMEM((16, D), jnp.float32),
        pltpu.VMEM_SHARED((N_ACC, D), jnp.float32),
        [pltpu.SemaphoreType.DMA for _ in range(4)],
        [pltpu.SemaphoreType.DMA for _ in range(4)],
        [pltpu.SemaphoreType.DMA for _ in range(2)],
        [pltpu.SemaphoreType.DMA for _ in range(2)],
    ]

    def body(src_hbm, dst_hbm, table_hbm, parts, srcv, dstv, rowbufs, zbuf,
             acc, isems, jsems, gsems, ssems):
        c = lax.axis_index("c")
        s = lax.axis_index("s")
        zero16 = jnp.zeros((16,), jnp.float32)

        for i in range(16):
            for j in range(D // 16):
                zbuf[i, pl.ds(j * 16, 16)] = zero16

        base0 = jnp.where(c == BIG_CORE,
                          s * (CH_BIG * K),
                          NS * (CH_BIG * K) + s * (CH_SML * K))

        idesc = [None] * CH_BIG
        jdesc = [None] * CH_BIG
        gdesc = [None] * CH_BIG
        sdesc = [None] * CH_BIG

        def load_idx(t):
            r = t % 4
            idesc[t] = pltpu.async_copy(
                src_hbm.at[pl.ds(base0 + t * K, K)], srcv[r], isems[r])
            jdesc[t] = pltpu.async_copy(
                dst_hbm.at[pl.ds(base0 + t * K, K)], dstv[r], jsems[r])

        load_idx(0)
        load_idx(1)

        for i in range(RPT // 16):
            pltpu.sync_copy(zbuf, acc.at[pl.ds(s * RPT + i * 16, 16), :])

        plsc.subcore_barrier()

        def iteration(t, bound):
            b = t % 2
            idesc[t].wait()
            jdesc[t].wait()
            if t >= 2:
                sdesc[t - 2].wait()
            gdesc[t] = pltpu.async_copy(table_hbm.at[srcv[t % 4]],
                                        rowbufs[b], gsems[b])
            if t + 2 < bound:
                load_idx(t + 2)
            if t >= 1:
                gdesc[t - 1].wait()
                sdesc[t - 1] = pltpu.async_copy(
                    rowbufs[1 - b], acc.at[dstv[(t - 1) % 4]],
                    ssems[1 - b], add=True)

        def epilogue(n):
            last = n - 1
            gdesc[last].wait()
            sdesc[last] = pltpu.async_copy(rowbufs[last % 2],
                                           acc.at[dstv[last % 4]],
                                           ssems[last % 2], add=True)
            sdesc[last - 1].wait()
            sdesc[last].wait()

        for t in range(CH_SML):
            iteration(t, CH_SML)

        @pl.when(c == BIG_CORE)
        def _big_core_tail():
            load_idx(CH_SML)
            load_idx(CH_SML + 1)
            for t in range(CH_SML, CH_BIG):
                iteration(t, CH_BIG)
            epilogue(CH_BIG)

        @pl.when(c != BIG_CORE)
        def _small_core_tail():
            epilogue(CH_SML)

        plsc.subcore_barrier()

        nout = RPT // K
        wdesc = [None] * nout
        for i in range(nout):
            b = i % 2
            r0 = s * RPT + i * K
            if i >= 2:
                wdesc[i - 2].wait()
            pltpu.sync_copy(acc.at[pl.ds(r0, K), :], rowbufs[b])
            wdesc[i] = pltpu.async_copy(rowbufs[b],
                                        parts.at[c, pl.ds(r0, K), :],
                                        gsems[b])
        for i in range(max(nout - 2, 0), nout):
            wdesc[i].wait()

    return pl.kernel(
        body,
        out_type=jax.ShapeDtypeStruct((NC, N_ACC, D), jnp.float32),
        mesh=plsc.VectorSubcoreMesh(core_axis_name="c", subcore_axis_name="s"),
        scratch_types=scratch,
    )


_seg_sum = _make_seg_kernel()
_cnt_sum = _make_cnt_kernel()


def _mm(a, b):
    return lax.dot_general(a, b, (((1,), (0,)), ((), ())),
                           precision=lax.Precision.HIGHEST,
                           preferred_element_type=jnp.float32)


def _inv_counts(cnt_blk):
    tot = cnt_blk[0, :, 0:1] + cnt_blk[1, :, 0:1]
    return 1.0 / jnp.maximum(tot, 1.0)


def _layer1_body(x_ref, p_ref, cnt_ref, ws1_ref, wn1_ref, ws2_ref, wn2_ref,
                 b1_ref, hs2_ref, hn2_ref):
    inv = _inv_counts(cnt_ref[...])
    agg = (p_ref[0] + p_ref[1]) * inv
    h1 = _mm(x_ref[...], ws1_ref[...]) + _mm(agg, wn1_ref[...]) + b1_ref[...]
    h1 = jnp.maximum(h1, 0.0)
    hs2_ref[...] = _mm(h1, ws2_ref[...])
    hn2_ref[...] = _mm(h1, wn2_ref[...])


def _layer2_body(hs2_ref, q_ref, cnt_ref, b2_ref, out_ref):
    inv = _inv_counts(cnt_ref[...])
    out_ref[...] = hs2_ref[...] + (q_ref[0] + q_ref[1]) * inv + b2_ref[...]


_row_spec = pl.BlockSpec((B, D), lambda i: (i, 0))
_part_spec = pl.BlockSpec((NC, B, D), lambda i: (0, i, 0))
_cnt_spec = pl.BlockSpec((NC, B, D), lambda i: (0, i, 0))
_w_spec = pl.BlockSpec((D, D), lambda i: (0, 0))
_b_spec = pl.BlockSpec((1, D), lambda i: (0, 0))

_layer1 = pl.pallas_call(
    _layer1_body,
    grid=(pl.cdiv(N, B),),
    in_specs=[_row_spec, _part_spec, _cnt_spec,
              _w_spec, _w_spec, _w_spec, _w_spec, _b_spec],
    out_specs=[_row_spec, _row_spec],
    out_shape=[jax.ShapeDtypeStruct((N, D), jnp.float32),
               jax.ShapeDtypeStruct((N, D), jnp.float32)],
)

_layer2 = pl.pallas_call(
    _layer2_body,
    grid=(pl.cdiv(N, B),),
    in_specs=[_row_spec, _part_spec, _cnt_spec, _b_spec],
    out_specs=_row_spec,
    out_shape=jax.ShapeDtypeStruct((N, D), jnp.float32),
)


def kernel(x, edge_index, W_self1, W_neigh1, b1, W_self2, W_neigh2, b2):
    src = edge_index[0]
    dst = edge_index[1]
    pad = E_PAD - E
    src_p = jnp.concatenate([src, jnp.zeros((pad,), jnp.int32)])
    dst_p = jnp.concatenate([dst, jnp.full((pad,), DUMMY, jnp.int32)])

    cnts = _cnt_sum(dst_p)
    seg1 = _seg_sum(src_p, dst_p, x)
    hs2, hn2 = _layer1(x, seg1, cnts, W_self1, W_neigh1, W_self2, W_neigh2,
                       b1.reshape(1, D))
    seg2 = _seg_sum(src_p, dst_p, hn2)
    return _layer2(hs2, seg2, cnts, b2.reshape(1, D))
